# baseline (speedup 1.0000x reference)
import functools

import jax
import jax.numpy as jnp
from jax import lax
from jax.experimental import pallas as pl
from jax.experimental.pallas import tpu as pltpu
from jax.experimental.pallas import tpu_sc as plsc

_T = 32


def kernel(inputs, pos_table):
    B, S, D = inputs.shape
    NC, NS = 2, 16
    NW = NC * NS
    seq_per_w = S // NW
    tiles = seq_per_w // _T
    n_steps = tiles * B
    mesh = plsc.VectorSubcoreMesh(core_axis_name="c", subcore_axis_name="s")

    @functools.partial(
        pl.kernel,
        mesh=mesh,
        out_type=jax.ShapeDtypeStruct((B, S, D), jnp.float32),
        scratch_types=[
            pltpu.VMEM((_T, D), jnp.float32),
            pltpu.VMEM((_T, D), jnp.float32),
            pltpu.VMEM((_T, D), jnp.float32),
            pltpu.VMEM((_T, D), jnp.float32),
            pltpu.SemaphoreType.DMA,
            pltpu.SemaphoreType.DMA,
            pltpu.SemaphoreType.DMA,
            pltpu.SemaphoreType.DMA,
            pltpu.SemaphoreType.DMA,
            pltpu.SemaphoreType.DMA,
        ],
    )
    def k(x_hbm, t_hbm, o_hbm, t_v, x0, x1, x2, l0, l1, l2, s0_, s1_, s2_):
        wid = lax.axis_index("s") * NC + lax.axis_index("c")
        base = wid * seq_per_w
        bufs = (x0, x1, x2)
        lsems = (l0, l1, l2)
        ssems = (s0_, s1_, s2_)

        def load(i):
            tile, b = divmod(i, B)
            row = base + tile * _T
            return pltpu.make_async_copy(
                x_hbm.at[b, pl.ds(row, _T)], bufs[i % 3], lsems[i % 3])

        def store(i):
            tile, b = divmod(i, B)
            row = base + tile * _T
            return pltpu.make_async_copy(
                bufs[i % 3], o_hbm.at[b, pl.ds(row, _T)], ssems[i % 3])

        load(0).start()
        for i in range(n_steps):
            if i + 1 < n_steps:
                if i - 2 >= 0:
                    store(i - 2).wait()
                load(i + 1).start()
            if i % B == 0:
                tile = i // B
                pltpu.sync_copy(t_hbm.at[pl.ds(base + tile * _T, _T)], t_v)
            load(i).wait()
            x_v = bufs[i % 3]

            def row_body(r, _):
                def col_body(c, _):
                    for j in range(16):
                        sl = pl.ds(c * 256 + j * 16, 16)
                        x_v[r, sl] = x_v[r, sl] + t_v[r, sl]
                    return 0

                lax.fori_loop(0, D // 256, col_body, 0)
                return 0

            lax.fori_loop(0, _T, row_body, 0)
            store(i).start()
        store(n_steps - 2).wait()
        store(n_steps - 1).wait()

    return k(inputs, pos_table)


# speedup vs baseline: 4.6929x; 4.6929x over previous
import jax
import jax.numpy as jnp
from jax.experimental import pallas as pl
from jax.experimental.pallas import tpu as pltpu

_BS = 1024


def _add_kernel(x_ref, t_ref, o_ref):
    o_ref[...] = x_ref[...] + t_ref[None]


def kernel(inputs, pos_table):
    B, S, D = inputs.shape
    return pl.pallas_call(
        _add_kernel,
        grid=(S // _BS,),
        in_specs=[
            pl.BlockSpec((B, _BS, D), lambda s: (0, s, 0)),
            pl.BlockSpec((_BS, D), lambda s: (s, 0)),
        ],
        out_specs=pl.BlockSpec((B, _BS, D), lambda s: (0, s, 0)),
        out_shape=jax.ShapeDtypeStruct((B, S, D), inputs.dtype),
        compiler_params=pltpu.CompilerParams(dimension_semantics=("parallel",)),
    )(inputs, pos_table)
